# baseline (speedup 1.0000x reference)
import functools
import math

import jax
import jax.numpy as jnp
from jax.experimental import pallas as pl
from jax.experimental.pallas import tpu as pltpu
from jax.experimental.pallas import tpu_sc as plsc

_NUM_CLASSES = 13
_BN_RSQRT = 1.0 / math.sqrt(1.0 + 1e-5)
_NSAMPLE = 32


def _fold(layer):
    W, b, g, be = layer
    s = g * _BN_RSQRT
    return (W * s[:, None]).T, b * s + be


def _fps_body(npoint, x_ref, y_ref, z_ref, cx_ref, cy_ref, cz_ref):
    X, Y, Z = x_ref[...], y_ref[...], z_ref[...]
    Bb, N = X.shape
    S = cx_ref.shape[1]
    iN = jax.lax.broadcasted_iota(jnp.int32, (Bb, N), 1)
    iS = jax.lax.broadcasted_iota(jnp.int32, (Bb, S), 1)

    def step(s, carry):
        dist, far, ox, oy, oz = carry
        m = iN == far
        cx = jnp.sum(jnp.where(m, X, 0.0), axis=1, keepdims=True)
        cy = jnp.sum(jnp.where(m, Y, 0.0), axis=1, keepdims=True)
        cz = jnp.sum(jnp.where(m, Z, 0.0), axis=1, keepdims=True)
        hit = iS == s
        ox = jnp.where(hit, cx, ox)
        oy = jnp.where(hit, cy, oy)
        oz = jnp.where(hit, cz, oz)
        dx = X - cx
        dy = Y - cy
        dz = Z - cz
        d = (dx * dx + dy * dy) + dz * dz
        dist = jnp.minimum(dist, d)
        dmax = jnp.max(dist, axis=1, keepdims=True)
        far = jnp.min(jnp.where(dist == dmax, iN, N), axis=1, keepdims=True)
        return dist, far, ox, oy, oz

    dist0 = jnp.full((Bb, N), 1e10, jnp.float32)
    far0 = jnp.zeros((Bb, 1), jnp.int32)
    o0 = jnp.zeros((Bb, S), jnp.float32)
    _, _, ox, oy, oz = jax.lax.fori_loop(0, npoint, step,
                                         (dist0, far0, o0, o0, o0))
    cx_ref[...] = ox
    cy_ref[...] = oy
    cz_ref[...] = oz


def _fps_pallas(xT, npoint):
    B, _, N = xT.shape
    out = jax.ShapeDtypeStruct((B, npoint), jnp.float32)
    cx, cy, cz = pl.pallas_call(
        functools.partial(_fps_body, npoint),
        in_specs=[pl.BlockSpec((B, N), lambda: (0, 0))] * 3,
        out_specs=[pl.BlockSpec((B, npoint), lambda: (0, 0))] * 3,
        out_shape=[out, out, out],
    )(xT[:, 0], xT[:, 1], xT[:, 2])
    return jnp.stack([cx, cy, cz], axis=-1)


def _ballq_body(radius, nsample, x_ref, y_ref, z_ref, c_ref, idx_ref):
    X, Y, Z = x_ref[0], y_ref[0], z_ref[0]
    C = c_ref[0]
    cx, cy, cz = C[:, 0:1], C[:, 1:2], C[:, 2:3]
    N = X.shape[1]
    s_src = (cx * cx + cy * cy) + cz * cz
    s_dst = (X * X + Y * Y) + Z * Z
    dot = (cx * X + cy * Y) + cz * Z
    sq = (s_src + s_dst) - 2.0 * dot
    mask = jnp.logical_not(sq > radius * radius)
    r = mask.astype(jnp.int16)
    sh = 1
    while sh < N:
        r = r + jnp.concatenate(
            [jnp.zeros(r.shape[:1] + (sh,), jnp.int16), r[:, :-sh]], axis=1)
        sh *= 2
    one = jnp.ones((), jnp.int16)
    zero = jnp.zeros((), jnp.int16)

    def count_le(k):
        v = jnp.where(r <= jnp.int16(k), one, zero)
        w = v.shape[1]
        while w > 128:
            w //= 2
            v = v[:, :w] + v[:, w:]
        return jnp.sum(v.astype(jnp.float32), axis=1, keepdims=True)

    cols = [count_le(k) for k in range(nsample)]
    out = jnp.concatenate(cols, axis=1).astype(jnp.int32)
    pad = out[:, 0:1]
    out = jnp.where(out == N, pad, out)
    idx_ref[0] = out + pl.program_id(0) * N


def _ballq_pallas(radius, nsample, xT, new_xyz):
    B, _, N = xT.shape
    S = new_xyz.shape[1]
    BS = min(S, 512)
    return pl.pallas_call(
        functools.partial(_ballq_body, radius, nsample),
        grid=(B, S // BS),
        in_specs=[
            pl.BlockSpec((1, 1, N), lambda b, s: (b, 0, 0)),
            pl.BlockSpec((1, 1, N), lambda b, s: (b, 0, 0)),
            pl.BlockSpec((1, 1, N), lambda b, s: (b, 0, 0)),
            pl.BlockSpec((1, BS, 3), lambda b, s: (b, s, 0)),
        ],
        out_specs=pl.BlockSpec((1, BS, nsample), lambda b, s: (b, s, 0)),
        out_shape=jax.ShapeDtypeStruct((B, S, nsample), jnp.int32),
        compiler_params=pltpu.CompilerParams(
            dimension_semantics=("parallel", "arbitrary")),
    )(xT[:, 0:1], xT[:, 1:2], xT[:, 2:3], new_xyz)


def _gact_body(p_ref, x_ref, wp_ref, wx_ref, b_ref, out_ref):
    P = p_ref[0]
    X = x_ref[0]
    out_ref[0] = (jnp.dot(P, wp_ref[...], preferred_element_type=jnp.float32)
                  + jnp.dot(X, wx_ref[...], preferred_element_type=jnp.float32)
                  + b_ref[...])


def _gact_pallas(points, xyz, WpT, WxT, b0):
    B, N, Cp = points.shape
    C1 = WpT.shape[1]
    return pl.pallas_call(
        _gact_body,
        grid=(B,),
        in_specs=[
            pl.BlockSpec((1, N, Cp), lambda b: (b, 0, 0)),
            pl.BlockSpec((1, N, 3), lambda b: (b, 0, 0)),
            pl.BlockSpec((Cp, C1), lambda b: (0, 0)),
            pl.BlockSpec((3, C1), lambda b: (0, 0)),
            pl.BlockSpec((1, C1), lambda b: (0, 0)),
        ],
        out_specs=pl.BlockSpec((1, N, C1), lambda b: (b, 0, 0)),
        out_shape=jax.ShapeDtypeStruct((B, N, C1), jnp.float32),
        compiler_params=pltpu.CompilerParams(
            dimension_semantics=("parallel",)),
    )(points, xyz, WpT, WxT, b0[None, :])


_SC_NC = 2
_SC_NS = 16


def _sc_gather(table, idx):
    R, C = table.shape
    M = idx.shape[0]
    NW = _SC_NC * _SC_NS
    b_per_w = M // NW
    chunk = min(b_per_w, max(128, 65536 // C))
    nchunks = b_per_w // chunk
    mesh = plsc.VectorSubcoreMesh(core_axis_name="c", subcore_axis_name="s")

    @functools.partial(
        pl.kernel,
        out_type=jax.ShapeDtypeStruct((M, C), jnp.float32),
        mesh=mesh,
        scratch_types=[
            pltpu.VMEM((chunk,), jnp.int32),
            pltpu.VMEM((chunk, C), jnp.float32),
            pltpu.SemaphoreType.DMA,
        ],
        compiler_params=pltpu.CompilerParams(use_tc_tiling_on_sc=False),
    )
    def gk(table_hbm, idx_hbm, out_hbm, idx_v, rows_v, sem):
        wid = jax.lax.axis_index("s") * _SC_NC + jax.lax.axis_index("c")
        base = wid * b_per_w
        for ci in range(nchunks):
            off = base + ci * chunk
            pltpu.sync_copy(idx_hbm.at[pl.ds(off, chunk)], idx_v)
            pltpu.async_copy(table_hbm.at[idx_v], rows_v, sem).wait()
            pltpu.sync_copy(rows_v, out_hbm.at[pl.ds(off, chunk)])

    return gk(table, idx)


def _samlp_body(g_ref, c_ref, wx_ref, w2_ref, b2_ref, w3_ref, b3_ref, out_ref):
    G = g_ref[0]
    BS, K, C1 = G.shape
    C = c_ref[0]
    wx = wx_ref[...]
    h = (C[:, 0:1] * wx[0:1] + C[:, 1:2] * wx[1:2]) + C[:, 2:3] * wx[2:3]
    act = jnp.maximum(G - h[:, None, :], 0.0)
    R = act.reshape(BS * K, C1)
    X2 = jnp.maximum(
        jnp.dot(R, w2_ref[...], preferred_element_type=jnp.float32)
        + b2_ref[...], 0.0)
    X3 = jnp.maximum(
        jnp.dot(X2, w3_ref[...], preferred_element_type=jnp.float32)
        + b3_ref[...], 0.0)
    C3 = X3.shape[1]
    out_ref[0] = jnp.max(X3.reshape(BS, K, C3), axis=1)


def _samlp_pallas(G, new_xyz, WxT, W2T, b2, W3T, b3):
    B, S, K, C1 = G.shape
    C2 = W2T.shape[1]
    C3 = W3T.shape[1]
    BS = min(S, 256)
    return pl.pallas_call(
        _samlp_body,
        grid=(B, S // BS),
        in_specs=[
            pl.BlockSpec((1, BS, K, C1), lambda b, s: (b, s, 0, 0)),
            pl.BlockSpec((1, BS, 3), lambda b, s: (b, s, 0)),
            pl.BlockSpec((3, C1), lambda b, s: (0, 0)),
            pl.BlockSpec((C1, C2), lambda b, s: (0, 0)),
            pl.BlockSpec((1, C2), lambda b, s: (0, 0)),
            pl.BlockSpec((C2, C3), lambda b, s: (0, 0)),
            pl.BlockSpec((1, C3), lambda b, s: (0, 0)),
        ],
        out_specs=pl.BlockSpec((1, BS, C3), lambda b, s: (b, s, 0)),
        out_shape=jax.ShapeDtypeStruct((B, S, C3), jnp.float32),
        compiler_params=pltpu.CompilerParams(
            dimension_semantics=("parallel", "arbitrary")),
    )(G, new_xyz, WxT, W2T, b2[None, :], W3T, b3[None, :])


def _fp_body(nlayers, has_p1, x1_ref, x2_ref, p2_ref, *rest):
    if has_p1:
        p1_ref = rest[0]
        rest = rest[1:]
    w_refs = rest[:-1]
    out_ref = rest[-1]

    X1 = x1_ref[0]
    X2 = x2_ref[0]
    P2 = p2_ref[0]
    BN = X1.shape[0]
    S = X2.shape[0]
    s1 = jnp.sum(X1 * X1, axis=1, keepdims=True)
    s2 = jnp.sum(X2 * X2, axis=1, keepdims=True)
    X1a = jnp.concatenate([-2.0 * X1, jnp.ones((BN, 1), jnp.float32)], axis=1)
    X2a = jnp.concatenate([X2, s2], axis=1)
    d = s1 + jax.lax.dot_general(X1a, X2a, (((1,), (1,)), ((), ())),
                                 preferred_element_type=jnp.float32)
    iS = jax.lax.broadcasted_iota(jnp.int32, (BN, S), 1)
    BIG = jnp.float32(3.0e38)

    def min3(dm):
        d1 = jnp.min(dm, axis=1, keepdims=True)
        i1 = jnp.min(jnp.where(dm == d1, iS, S), axis=1, keepdims=True)
        return d1, i1

    d1, i1 = min3(d)
    dmask = jnp.where(iS == i1, BIG, d)
    d2, i2 = min3(dmask)
    dmask = jnp.where(iS == i2, BIG, dmask)
    d3, i3 = min3(dmask)
    r1 = 1.0 / (d1 + 1e-8)
    r2 = 1.0 / (d2 + 1e-8)
    r3 = 1.0 / (d3 + 1e-8)
    tot = (r1 + r2) + r3
    Rm = (jnp.where(iS == i1, r1 / tot, 0.0)
          + jnp.where(iS == i2, r2 / tot, 0.0)
          + jnp.where(iS == i3, r3 / tot, 0.0))
    interp = jnp.dot(Rm, P2, preferred_element_type=jnp.float32)

    wi = 0
    if has_p1:
        X = (jnp.dot(p1_ref[0], w_refs[0][...],
                     preferred_element_type=jnp.float32)
             + jnp.dot(interp, w_refs[1][...],
                       preferred_element_type=jnp.float32)
             + w_refs[2][...])
        wi = 3
    else:
        X = (jnp.dot(interp, w_refs[0][...],
                     preferred_element_type=jnp.float32) + w_refs[1][...])
        wi = 2
    X = jnp.maximum(X, 0.0)
    for _ in range(nlayers - 1):
        X = jnp.maximum(
            jnp.dot(X, w_refs[wi][...], preferred_element_type=jnp.float32)
            + w_refs[wi + 1][...], 0.0)
        wi += 2
    out_ref[0] = X


def _fp_pallas(xyz1, xyz2, points1, points2, layers):
    B, N, _ = xyz1.shape
    S = xyz2.shape[1]
    C2 = points2.shape[2]
    BN = min(N, 1024)
    folded = [_fold(l) for l in layers]
    W0T, b0 = folded[0]
    weights = []
    wspecs = []

    def const_spec(a):
        weights.append(a)
        shp = a.shape
        wspecs.append(pl.BlockSpec(shp, lambda b, n: (0,) * len(shp)))

    has_p1 = points1 is not None
    if has_p1:
        Cp = points1.shape[2]
        const_spec(W0T[:Cp])
        const_spec(W0T[Cp:])
        const_spec(b0[None, :])
    else:
        const_spec(W0T)
        const_spec(b0[None, :])
    for WT, b in folded[1:]:
        const_spec(WT)
        const_spec(b[None, :])
    Cout = folded[-1][1].shape[0]

    in_specs = [
        pl.BlockSpec((1, BN, 3), lambda b, n: (b, n, 0)),
        pl.BlockSpec((1, S, 3), lambda b, n: (b, 0, 0)),
        pl.BlockSpec((1, S, C2), lambda b, n: (b, 0, 0)),
    ]
    args = [xyz1, xyz2, points2]
    if has_p1:
        in_specs.append(pl.BlockSpec((1, BN, Cp), lambda b, n: (b, n, 0)))
        args.append(points1)
    in_specs += wspecs
    args += weights
    return pl.pallas_call(
        functools.partial(_fp_body, len(layers), has_p1),
        grid=(B, N // BN),
        in_specs=in_specs,
        out_specs=pl.BlockSpec((1, BN, Cout), lambda b, n: (b, n, 0)),
        out_shape=jax.ShapeDtypeStruct((B, N, Cout), jnp.float32),
        compiler_params=pltpu.CompilerParams(
            dimension_semantics=("parallel", "arbitrary")),
    )(*args)


def _head_body(feat_ref, w1_ref, b1_ref, w2_ref, b2_ref, out_ref):
    feat = feat_ref[0]
    h = jnp.dot(w1_ref[...], feat, preferred_element_type=jnp.float32) + b1_ref[...]
    h = jnp.maximum(h, 0.0)
    logits = jnp.dot(w2_ref[...], h, preferred_element_type=jnp.float32) + b2_ref[...]
    m = jnp.max(logits, axis=0, keepdims=True)
    z = logits - m
    lse = jnp.log(jnp.sum(jnp.exp(z), axis=0, keepdims=True))
    out_ref[0] = z - lse


def _head(featT, params):
    B, C, N = featT.shape
    W1, b1, g1, be1 = params['conv1']
    s1 = g1 * _BN_RSQRT
    w1f = W1 * s1[:, None]
    b1f = (b1 * s1 + be1)[:, None]
    W2, b2 = params['conv2']
    b2f = b2[:, None]
    return pl.pallas_call(
        _head_body,
        grid=(B,),
        in_specs=[
            pl.BlockSpec((1, C, N), lambda b: (b, 0, 0)),
            pl.BlockSpec((C, C), lambda b: (0, 0)),
            pl.BlockSpec((C, 1), lambda b: (0, 0)),
            pl.BlockSpec((_NUM_CLASSES, C), lambda b: (0, 0)),
            pl.BlockSpec((_NUM_CLASSES, 1), lambda b: (0, 0)),
        ],
        out_specs=pl.BlockSpec((1, _NUM_CLASSES, N), lambda b: (b, 0, 0)),
        out_shape=jax.ShapeDtypeStruct((B, _NUM_CLASSES, N), jnp.float32),
        compiler_params=pltpu.CompilerParams(
            dimension_semantics=("parallel",)),
    )(featT, w1f, b1f, W2, b2f)


def _sa_stage(xT, xyz_rm, points_rm, npoint, radius, layers):
    B, _, N = xT.shape
    new_xyz = _fps_pallas(xT, npoint)
    idx = _ballq_pallas(radius, _NSAMPLE, xT, new_xyz)
    W0T, b0 = _fold(layers[0])
    WxT, WpT = W0T[:3], W0T[3:]
    C1 = W0T.shape[1]
    g = _gact_pallas(points_rm, xyz_rm, WpT, WxT, b0)
    rows = _sc_gather(g.reshape(B * N, C1),
                      idx.reshape(B * npoint * _NSAMPLE))
    W2T, b2 = _fold(layers[1])
    W3T, b3 = _fold(layers[2])
    pooled = _samlp_pallas(rows.reshape(B, npoint, _NSAMPLE, C1),
                           new_xyz, WxT, W2T, b2, W3T, b3)
    return new_xyz, pooled


def kernel(data, params):
    xT0 = data[:, :3, :]
    l0_xyz = jnp.transpose(xT0, (0, 2, 1))

    l1_xyz, l1_points = _sa_stage(xT0, l0_xyz, l0_xyz, 1024, 0.1,
                                  params['sa1'])
    xT1 = jnp.transpose(l1_xyz, (0, 2, 1))
    l2_xyz, l2_points = _sa_stage(xT1, l1_xyz, l1_points, 256, 0.2,
                                  params['sa2'])
    xT2 = jnp.transpose(l2_xyz, (0, 2, 1))
    l3_xyz, l3_points = _sa_stage(xT2, l2_xyz, l2_points, 64, 0.4,
                                  params['sa3'])
    xT3 = jnp.transpose(l3_xyz, (0, 2, 1))
    l4_xyz, l4_points = _sa_stage(xT3, l3_xyz, l3_points, 16, 0.8,
                                  params['sa4'])

    l3_points = _fp_pallas(l3_xyz, l4_xyz, l3_points, l4_points, params['fp4'])
    l2_points = _fp_pallas(l2_xyz, l3_xyz, l2_points, l3_points, params['fp3'])
    l1_points = _fp_pallas(l1_xyz, l2_xyz, l1_points, l2_points, params['fp2'])
    l0_feat = _fp_pallas(l0_xyz, l1_xyz, None, l1_points, params['fp1'])

    featT = jnp.transpose(l0_feat, (0, 2, 1))
    return _head(featT, params)


# speedup vs baseline: 1.0062x; 1.0062x over previous
import functools
import math

import jax
import jax.numpy as jnp
from jax.experimental import pallas as pl
from jax.experimental.pallas import tpu as pltpu
from jax.experimental.pallas import tpu_sc as plsc

_NUM_CLASSES = 13
_BN_RSQRT = 1.0 / math.sqrt(1.0 + 1e-5)
_NSAMPLE = 32


def _fold(layer):
    W, b, g, be = layer
    s = g * _BN_RSQRT
    return (W * s[:, None]).T, b * s + be


def _fps_body(npoint, x_ref, y_ref, z_ref, cx_ref, cy_ref, cz_ref):
    X, Y, Z = x_ref[...], y_ref[...], z_ref[...]
    Bb, N = X.shape
    S = cx_ref.shape[1]
    iN = jax.lax.broadcasted_iota(jnp.int32, (Bb, N), 1)
    iS = jax.lax.broadcasted_iota(jnp.int32, (Bb, S), 1)

    def step(s, carry):
        dist, far, ox, oy, oz = carry
        m = iN == far
        cx = jnp.sum(jnp.where(m, X, 0.0), axis=1, keepdims=True)
        cy = jnp.sum(jnp.where(m, Y, 0.0), axis=1, keepdims=True)
        cz = jnp.sum(jnp.where(m, Z, 0.0), axis=1, keepdims=True)
        hit = iS == s
        ox = jnp.where(hit, cx, ox)
        oy = jnp.where(hit, cy, oy)
        oz = jnp.where(hit, cz, oz)
        dx = X - cx
        dy = Y - cy
        dz = Z - cz
        d = (dx * dx + dy * dy) + dz * dz
        dist = jnp.minimum(dist, d)
        dmax = jnp.max(dist, axis=1, keepdims=True)
        far = jnp.min(jnp.where(dist == dmax, iN, N), axis=1, keepdims=True)
        return dist, far, ox, oy, oz

    dist0 = jnp.full((Bb, N), 1e10, jnp.float32)
    far0 = jnp.zeros((Bb, 1), jnp.int32)
    o0 = jnp.zeros((Bb, S), jnp.float32)
    _, _, ox, oy, oz = jax.lax.fori_loop(0, npoint, step,
                                         (dist0, far0, o0, o0, o0))
    cx_ref[...] = ox
    cy_ref[...] = oy
    cz_ref[...] = oz


def _fps_pallas(xT, npoint):
    B, _, N = xT.shape
    out = jax.ShapeDtypeStruct((B, npoint), jnp.float32)
    cx, cy, cz = pl.pallas_call(
        functools.partial(_fps_body, npoint),
        in_specs=[pl.BlockSpec((B, N), lambda: (0, 0))] * 3,
        out_specs=[pl.BlockSpec((B, npoint), lambda: (0, 0))] * 3,
        out_shape=[out, out, out],
    )(xT[:, 0], xT[:, 1], xT[:, 2])
    return jnp.stack([cx, cy, cz], axis=-1)


def _ballq_body(radius, nsample, x_ref, y_ref, z_ref, c_ref, idx_ref):
    X, Y, Z = x_ref[0], y_ref[0], z_ref[0]
    C = c_ref[0]
    cx, cy, cz = C[:, 0:1], C[:, 1:2], C[:, 2:3]
    N = X.shape[1]
    s_src = (cx * cx + cy * cy) + cz * cz
    s_dst = (X * X + Y * Y) + Z * Z
    dot = (cx * X + cy * Y) + cz * Z
    sq = (s_src + s_dst) - 2.0 * dot
    mask = jnp.logical_not(sq > radius * radius)
    r = mask.astype(jnp.int16)
    sh = 1
    while sh < N:
        r = r + jnp.concatenate(
            [jnp.zeros(r.shape[:1] + (sh,), jnp.int16), r[:, :-sh]], axis=1)
        sh *= 2
    one = jnp.ones((), jnp.int16)
    zero = jnp.zeros((), jnp.int16)

    def count_le(k):
        v = jnp.where(r <= jnp.int16(k), one, zero)
        w = v.shape[1]
        while w > 128:
            w //= 2
            v = v[:, :w] + v[:, w:]
        return jnp.sum(v.astype(jnp.float32), axis=1, keepdims=True)

    cols = [count_le(k) for k in range(nsample)]
    out = jnp.concatenate(cols, axis=1).astype(jnp.int32)
    pad = out[:, 0:1]
    out = jnp.where(out == N, pad, out)
    idx_ref[0] = out + pl.program_id(0) * N


def _ballq_pallas(radius, nsample, xT, new_xyz):
    B, _, N = xT.shape
    S = new_xyz.shape[1]
    BS = min(S, 256)
    return pl.pallas_call(
        functools.partial(_ballq_body, radius, nsample),
        grid=(B, S // BS),
        in_specs=[
            pl.BlockSpec((1, 1, N), lambda b, s: (b, 0, 0)),
            pl.BlockSpec((1, 1, N), lambda b, s: (b, 0, 0)),
            pl.BlockSpec((1, 1, N), lambda b, s: (b, 0, 0)),
            pl.BlockSpec((1, BS, 3), lambda b, s: (b, s, 0)),
        ],
        out_specs=pl.BlockSpec((1, BS, nsample), lambda b, s: (b, s, 0)),
        out_shape=jax.ShapeDtypeStruct((B, S, nsample), jnp.int32),
        compiler_params=pltpu.CompilerParams(
            dimension_semantics=("parallel", "arbitrary")),
    )(xT[:, 0:1], xT[:, 1:2], xT[:, 2:3], new_xyz)


def _gact_body(p_ref, x_ref, wp_ref, wx_ref, b_ref, out_ref):
    P = p_ref[0]
    X = x_ref[0]
    out_ref[0] = (jnp.dot(P, wp_ref[...], preferred_element_type=jnp.float32)
                  + jnp.dot(X, wx_ref[...], preferred_element_type=jnp.float32)
                  + b_ref[...])


def _gact_pallas(points, xyz, WpT, WxT, b0):
    B, N, Cp = points.shape
    C1 = WpT.shape[1]
    return pl.pallas_call(
        _gact_body,
        grid=(B,),
        in_specs=[
            pl.BlockSpec((1, N, Cp), lambda b: (b, 0, 0)),
            pl.BlockSpec((1, N, 3), lambda b: (b, 0, 0)),
            pl.BlockSpec((Cp, C1), lambda b: (0, 0)),
            pl.BlockSpec((3, C1), lambda b: (0, 0)),
            pl.BlockSpec((1, C1), lambda b: (0, 0)),
        ],
        out_specs=pl.BlockSpec((1, N, C1), lambda b: (b, 0, 0)),
        out_shape=jax.ShapeDtypeStruct((B, N, C1), jnp.float32),
        compiler_params=pltpu.CompilerParams(
            dimension_semantics=("parallel",)),
    )(points, xyz, WpT, WxT, b0[None, :])


_SC_NC = 2
_SC_NS = 16


def _sc_gather(table, idx):
    R, C = table.shape
    M = idx.shape[0]
    NW = _SC_NC * _SC_NS
    b_per_w = M // NW
    chunk = min(b_per_w, max(128, 65536 // C))
    nchunks = b_per_w // chunk
    mesh = plsc.VectorSubcoreMesh(core_axis_name="c", subcore_axis_name="s")

    @functools.partial(
        pl.kernel,
        out_type=jax.ShapeDtypeStruct((M, C), jnp.float32),
        mesh=mesh,
        scratch_types=[
            pltpu.VMEM((chunk,), jnp.int32),
            pltpu.VMEM((chunk, C), jnp.float32),
            pltpu.SemaphoreType.DMA,
        ],
        compiler_params=pltpu.CompilerParams(use_tc_tiling_on_sc=False),
    )
    def gk(table_hbm, idx_hbm, out_hbm, idx_v, rows_v, sem):
        wid = jax.lax.axis_index("s") * _SC_NC + jax.lax.axis_index("c")
        base = wid * b_per_w
        for ci in range(nchunks):
            off = base + ci * chunk
            pltpu.sync_copy(idx_hbm.at[pl.ds(off, chunk)], idx_v)
            pltpu.async_copy(table_hbm.at[idx_v], rows_v, sem).wait()
            pltpu.sync_copy(rows_v, out_hbm.at[pl.ds(off, chunk)])

    return gk(table, idx)


def _samlp_body(g_ref, c_ref, wx_ref, w2_ref, b2_ref, w3_ref, b3_ref, out_ref):
    G = g_ref[0]
    BS, K, C1 = G.shape
    C = c_ref[0]
    wx = wx_ref[...]
    h = (C[:, 0:1] * wx[0:1] + C[:, 1:2] * wx[1:2]) + C[:, 2:3] * wx[2:3]
    act = jnp.maximum(G - h[:, None, :], 0.0)
    R = act.reshape(BS * K, C1)
    X2 = jnp.maximum(
        jnp.dot(R, w2_ref[...], preferred_element_type=jnp.float32)
        + b2_ref[...], 0.0)
    X3 = jnp.maximum(
        jnp.dot(X2, w3_ref[...], preferred_element_type=jnp.float32)
        + b3_ref[...], 0.0)
    C3 = X3.shape[1]
    out_ref[0] = jnp.max(X3.reshape(BS, K, C3), axis=1)


def _samlp_pallas(G, new_xyz, WxT, W2T, b2, W3T, b3):
    B, S, K, C1 = G.shape
    C2 = W2T.shape[1]
    C3 = W3T.shape[1]
    BS = min(S, 256)
    return pl.pallas_call(
        _samlp_body,
        grid=(B, S // BS),
        in_specs=[
            pl.BlockSpec((1, BS, K, C1), lambda b, s: (b, s, 0, 0)),
            pl.BlockSpec((1, BS, 3), lambda b, s: (b, s, 0)),
            pl.BlockSpec((3, C1), lambda b, s: (0, 0)),
            pl.BlockSpec((C1, C2), lambda b, s: (0, 0)),
            pl.BlockSpec((1, C2), lambda b, s: (0, 0)),
            pl.BlockSpec((C2, C3), lambda b, s: (0, 0)),
            pl.BlockSpec((1, C3), lambda b, s: (0, 0)),
        ],
        out_specs=pl.BlockSpec((1, BS, C3), lambda b, s: (b, s, 0)),
        out_shape=jax.ShapeDtypeStruct((B, S, C3), jnp.float32),
        compiler_params=pltpu.CompilerParams(
            dimension_semantics=("parallel", "arbitrary")),
    )(G, new_xyz, WxT, W2T, b2[None, :], W3T, b3[None, :])


def _fp_body(nlayers, has_p1, x1_ref, x2_ref, p2_ref, *rest):
    if has_p1:
        p1_ref = rest[0]
        rest = rest[1:]
    w_refs = rest[:-1]
    out_ref = rest[-1]

    X1 = x1_ref[0]
    X2 = x2_ref[0]
    P2 = p2_ref[0]
    BN = X1.shape[0]
    S = X2.shape[0]
    s1 = jnp.sum(X1 * X1, axis=1, keepdims=True)
    s2 = jnp.sum(X2 * X2, axis=1, keepdims=True)
    X1a = jnp.concatenate([-2.0 * X1, jnp.ones((BN, 1), jnp.float32)], axis=1)
    X2a = jnp.concatenate([X2, s2], axis=1)
    d = s1 + jax.lax.dot_general(X1a, X2a, (((1,), (1,)), ((), ())),
                                 preferred_element_type=jnp.float32)
    iS = jax.lax.broadcasted_iota(jnp.int32, (BN, S), 1)
    BIG = jnp.float32(3.0e38)

    def min3(dm):
        d1 = jnp.min(dm, axis=1, keepdims=True)
        i1 = jnp.min(jnp.where(dm == d1, iS, S), axis=1, keepdims=True)
        return d1, i1

    d1, i1 = min3(d)
    dmask = jnp.where(iS == i1, BIG, d)
    d2, i2 = min3(dmask)
    dmask = jnp.where(iS == i2, BIG, dmask)
    d3, i3 = min3(dmask)
    r1 = 1.0 / (d1 + 1e-8)
    r2 = 1.0 / (d2 + 1e-8)
    r3 = 1.0 / (d3 + 1e-8)
    tot = (r1 + r2) + r3
    Rm = (jnp.where(iS == i1, r1 / tot, 0.0)
          + jnp.where(iS == i2, r2 / tot, 0.0)
          + jnp.where(iS == i3, r3 / tot, 0.0))
    interp = jnp.dot(Rm, P2, preferred_element_type=jnp.float32)

    wi = 0
    if has_p1:
        X = (jnp.dot(p1_ref[0], w_refs[0][...],
                     preferred_element_type=jnp.float32)
             + jnp.dot(interp, w_refs[1][...],
                       preferred_element_type=jnp.float32)
             + w_refs[2][...])
        wi = 3
    else:
        X = (jnp.dot(interp, w_refs[0][...],
                     preferred_element_type=jnp.float32) + w_refs[1][...])
        wi = 2
    X = jnp.maximum(X, 0.0)
    for _ in range(nlayers - 1):
        X = jnp.maximum(
            jnp.dot(X, w_refs[wi][...], preferred_element_type=jnp.float32)
            + w_refs[wi + 1][...], 0.0)
        wi += 2
    out_ref[0] = X


def _fp_pallas(xyz1, xyz2, points1, points2, layers):
    B, N, _ = xyz1.shape
    S = xyz2.shape[1]
    C2 = points2.shape[2]
    BN = min(N, 1024)
    folded = [_fold(l) for l in layers]
    W0T, b0 = folded[0]
    weights = []
    wspecs = []

    def const_spec(a):
        weights.append(a)
        shp = a.shape
        wspecs.append(pl.BlockSpec(shp, lambda b, n: (0,) * len(shp)))

    has_p1 = points1 is not None
    if has_p1:
        Cp = points1.shape[2]
        const_spec(W0T[:Cp])
        const_spec(W0T[Cp:])
        const_spec(b0[None, :])
    else:
        const_spec(W0T)
        const_spec(b0[None, :])
    for WT, b in folded[1:]:
        const_spec(WT)
        const_spec(b[None, :])
    Cout = folded[-1][1].shape[0]

    in_specs = [
        pl.BlockSpec((1, BN, 3), lambda b, n: (b, n, 0)),
        pl.BlockSpec((1, S, 3), lambda b, n: (b, 0, 0)),
        pl.BlockSpec((1, S, C2), lambda b, n: (b, 0, 0)),
    ]
    args = [xyz1, xyz2, points2]
    if has_p1:
        in_specs.append(pl.BlockSpec((1, BN, Cp), lambda b, n: (b, n, 0)))
        args.append(points1)
    in_specs += wspecs
    args += weights
    return pl.pallas_call(
        functools.partial(_fp_body, len(layers), has_p1),
        grid=(B, N // BN),
        in_specs=in_specs,
        out_specs=pl.BlockSpec((1, BN, Cout), lambda b, n: (b, n, 0)),
        out_shape=jax.ShapeDtypeStruct((B, N, Cout), jnp.float32),
        compiler_params=pltpu.CompilerParams(
            dimension_semantics=("parallel", "arbitrary")),
    )(*args)


def _head_body(feat_ref, w1_ref, b1_ref, w2_ref, b2_ref, out_ref):
    feat = feat_ref[0]
    h = jnp.dot(w1_ref[...], feat, preferred_element_type=jnp.float32) + b1_ref[...]
    h = jnp.maximum(h, 0.0)
    logits = jnp.dot(w2_ref[...], h, preferred_element_type=jnp.float32) + b2_ref[...]
    m = jnp.max(logits, axis=0, keepdims=True)
    z = logits - m
    lse = jnp.log(jnp.sum(jnp.exp(z), axis=0, keepdims=True))
    out_ref[0] = z - lse


def _head(featT, params):
    B, C, N = featT.shape
    W1, b1, g1, be1 = params['conv1']
    s1 = g1 * _BN_RSQRT
    w1f = W1 * s1[:, None]
    b1f = (b1 * s1 + be1)[:, None]
    W2, b2 = params['conv2']
    b2f = b2[:, None]
    return pl.pallas_call(
        _head_body,
        grid=(B,),
        in_specs=[
            pl.BlockSpec((1, C, N), lambda b: (b, 0, 0)),
            pl.BlockSpec((C, C), lambda b: (0, 0)),
            pl.BlockSpec((C, 1), lambda b: (0, 0)),
            pl.BlockSpec((_NUM_CLASSES, C), lambda b: (0, 0)),
            pl.BlockSpec((_NUM_CLASSES, 1), lambda b: (0, 0)),
        ],
        out_specs=pl.BlockSpec((1, _NUM_CLASSES, N), lambda b: (b, 0, 0)),
        out_shape=jax.ShapeDtypeStruct((B, _NUM_CLASSES, N), jnp.float32),
        compiler_params=pltpu.CompilerParams(
            dimension_semantics=("parallel",)),
    )(featT, w1f, b1f, W2, b2f)


def _sa_stage(xT, xyz_rm, points_rm, npoint, radius, layers):
    B, _, N = xT.shape
    new_xyz = _fps_pallas(xT, npoint)
    idx = _ballq_pallas(radius, _NSAMPLE, xT, new_xyz)
    W0T, b0 = _fold(layers[0])
    WxT, WpT = W0T[:3], W0T[3:]
    C1 = W0T.shape[1]
    g = _gact_pallas(points_rm, xyz_rm, WpT, WxT, b0)
    rows = _sc_gather(g.reshape(B * N, C1),
                      idx.reshape(B * npoint * _NSAMPLE))
    W2T, b2 = _fold(layers[1])
    W3T, b3 = _fold(layers[2])
    pooled = _samlp_pallas(rows.reshape(B, npoint, _NSAMPLE, C1),
                           new_xyz, WxT, W2T, b2, W3T, b3)
    return new_xyz, pooled


def kernel(data, params):
    xT0 = data[:, :3, :]
    l0_xyz = jnp.transpose(xT0, (0, 2, 1))

    l1_xyz, l1_points = _sa_stage(xT0, l0_xyz, l0_xyz, 1024, 0.1,
                                  params['sa1'])
    xT1 = jnp.transpose(l1_xyz, (0, 2, 1))
    l2_xyz, l2_points = _sa_stage(xT1, l1_xyz, l1_points, 256, 0.2,
                                  params['sa2'])
    xT2 = jnp.transpose(l2_xyz, (0, 2, 1))
    l3_xyz, l3_points = _sa_stage(xT2, l2_xyz, l2_points, 64, 0.4,
                                  params['sa3'])
    xT3 = jnp.transpose(l3_xyz, (0, 2, 1))
    l4_xyz, l4_points = _sa_stage(xT3, l3_xyz, l3_points, 16, 0.8,
                                  params['sa4'])

    l3_points = _fp_pallas(l3_xyz, l4_xyz, l3_points, l4_points, params['fp4'])
    l2_points = _fp_pallas(l2_xyz, l3_xyz, l2_points, l3_points, params['fp3'])
    l1_points = _fp_pallas(l1_xyz, l2_xyz, l1_points, l2_points, params['fp2'])
    l0_feat = _fp_pallas(l0_xyz, l1_xyz, None, l1_points, params['fp1'])

    featT = jnp.transpose(l0_feat, (0, 2, 1))
    return _head(featT, params)
